# baseline (speedup 1.0000x reference)
import jax
import jax.numpy as jnp
from jax.experimental import pallas as pl
from jax.experimental.pallas import tpu as pltpu

_BLOCK_ROWS = 256


def _scale_body(w_ref, x_ref, o_ref):
    f = w_ref[0, 0]
    o_ref[...] = x_ref[...] * (f * f)


def kernel(x, multiple_factor_weight):
    b, r, c = x.shape
    x2d = x.reshape(b * r, c)
    n_rows = b * r
    grid = (n_rows // _BLOCK_ROWS,)
    out = pl.pallas_call(
        _scale_body,
        grid=grid,
        in_specs=[
            pl.BlockSpec(memory_space=pltpu.MemorySpace.SMEM),
            pl.BlockSpec((_BLOCK_ROWS, c), lambda i: (i, 0)),
        ],
        out_specs=pl.BlockSpec((_BLOCK_ROWS, c), lambda i: (i, 0)),
        out_shape=jax.ShapeDtypeStruct((n_rows, c), x.dtype),
        compiler_params=pltpu.CompilerParams(
            dimension_semantics=("parallel",),
        ),
    )(multiple_factor_weight, x2d)
    return out.reshape(b, r, c)


# speedup vs baseline: 1.0134x; 1.0134x over previous
import jax
import jax.numpy as jnp
from jax.experimental import pallas as pl
from jax.experimental.pallas import tpu as pltpu

_BLOCK_ROWS = 512


def _scale_body(w_ref, x_ref, o_ref):
    f = w_ref[0, 0]
    o_ref[...] = x_ref[...] * (f * f)


def kernel(x, multiple_factor_weight):
    b, r, c = x.shape
    x2d = x.reshape(b * r, c)
    n_rows = b * r
    grid = (n_rows // _BLOCK_ROWS,)
    out = pl.pallas_call(
        _scale_body,
        grid=grid,
        in_specs=[
            pl.BlockSpec(memory_space=pltpu.MemorySpace.SMEM),
            pl.BlockSpec((_BLOCK_ROWS, c), lambda i: (i, 0)),
        ],
        out_specs=pl.BlockSpec((_BLOCK_ROWS, c), lambda i: (i, 0)),
        out_shape=jax.ShapeDtypeStruct((n_rows, c), x.dtype),
        compiler_params=pltpu.CompilerParams(
            dimension_semantics=("parallel",),
        ),
    )(multiple_factor_weight, x2d)
    return out.reshape(b, r, c)


# speedup vs baseline: 1.0155x; 1.0021x over previous
import jax
import jax.numpy as jnp
from jax.experimental import pallas as pl
from jax.experimental.pallas import tpu as pltpu

_BLOCK_ROWS = 512
_NBUF = 3


def _stream_body(w_ref, x_hbm, o_hbm, in_buf, out_buf, in_sem, out_sem):
    f = w_ref[0, 0]
    f2 = f * f
    n_chunks = x_hbm.shape[0] // _BLOCK_ROWS

    def in_copy(i, slot):
        return pltpu.make_async_copy(
            x_hbm.at[pl.ds(i * _BLOCK_ROWS, _BLOCK_ROWS), :],
            in_buf.at[slot], in_sem.at[slot])

    def out_copy(i, slot):
        return pltpu.make_async_copy(
            out_buf.at[slot],
            o_hbm.at[pl.ds(i * _BLOCK_ROWS, _BLOCK_ROWS), :], out_sem.at[slot])

    for i in range(min(_NBUF, n_chunks)):
        in_copy(i, i).start()
    for i in range(n_chunks):
        slot = i % _NBUF
        in_copy(i, slot).wait()
        if i >= _NBUF:
            out_copy(i - _NBUF, slot).wait()
        out_buf[slot] = in_buf[slot] * f2
        out_copy(i, slot).start()
        if i + _NBUF < n_chunks:
            in_copy(i + _NBUF, slot).start()
    for i in range(max(0, n_chunks - _NBUF), n_chunks):
        out_copy(i, i % _NBUF).wait()


def kernel(x, multiple_factor_weight):
    b, r, c = x.shape
    n_rows = b * r
    x2d = x.reshape(n_rows, c)
    out = pl.pallas_call(
        _stream_body,
        in_specs=[
            pl.BlockSpec(memory_space=pltpu.MemorySpace.SMEM),
            pl.BlockSpec(memory_space=pltpu.MemorySpace.HBM),
        ],
        out_specs=pl.BlockSpec(memory_space=pltpu.MemorySpace.HBM),
        out_shape=jax.ShapeDtypeStruct((n_rows, c), x.dtype),
        scratch_shapes=[
            pltpu.VMEM((_NBUF, _BLOCK_ROWS, c), jnp.float32),
            pltpu.VMEM((_NBUF, _BLOCK_ROWS, c), jnp.float32),
            pltpu.SemaphoreType.DMA((_NBUF,)),
            pltpu.SemaphoreType.DMA((_NBUF,)),
        ],
    )(multiple_factor_weight, x2d)
    return out.reshape(b, r, c)
